# Initial kernel scaffold; baseline (speedup 1.0000x reference)
import jax
import jax.numpy as jnp
from jax.experimental import pallas as pl


def kernel(user_fea, item_fea, global_embedding_u, global_embedding_i, W, b, edge_weight, edge_index):
    raise NotImplementedError("write your pallas kernel here")


# speedup vs baseline: 3.8723x; 3.8723x over previous
import dataclasses
import functools

import jax
import jax.numpy as jnp
from jax import lax
from jax.experimental import pallas as pl
from jax.experimental.pallas import tpu as pltpu
from jax.experimental.pallas import tpu_sc as plsc

NC = 2
NS = 16
L = 16
CH = 128
N_LAYERS = 4


def _embed_call(user_fea, item_fea, gu, gi, W, b):
    nu, d = user_fea.shape
    ni = item_fea.shape[0]

    def body(uf, itf, gu_r, gi_r, w_r, b_r, out):
        gus = jnp.sum(gu_r[...], axis=0)
        gis = jnp.sum(gi_r[...], axis=0)
        out[:nu] = uf[...] + gus[None, :]
        acts = jnp.dot(itf[...], w_r[...],
                       preferred_element_type=jnp.float32,
                       precision=lax.Precision.HIGHEST)
        out[nu:] = jnp.tanh(acts + b_r[...][None, :]) + gis[None, :]

    return pl.pallas_call(
        body,
        out_shape=jax.ShapeDtypeStruct((nu + ni, d), jnp.float32),
    )(user_fea, item_fea, gu, gi, W, b)


def _cosine_call(parts, ego, ui):
    n, d = ego.shape

    def body(p, e_r, u_r, new_r, uo_r):
        x = p[0] + p[1]
        e = e_r[...]
        num = jnp.sum(x * e, axis=-1, keepdims=True)
        nx = jnp.sqrt(jnp.sum(x * x, axis=-1, keepdims=True))
        ne = jnp.sqrt(jnp.sum(e * e, axis=-1, keepdims=True))
        wgt = num / jnp.maximum(nx * ne, 1e-8)
        xw = x * wgt
        new_r[...] = xw
        uo_r[...] = u_r[...] + xw

    return pl.pallas_call(
        body,
        out_shape=(jax.ShapeDtypeStruct((n, d), jnp.float32),
                   jax.ShapeDtypeStruct((n, d), jnp.float32)),
    )(parts, ego, ui)


@functools.lru_cache(maxsize=None)
def _make_spmm(n_nodes, d, t_chunks):
    rpt = (n_nodes // (NS * 8)) * 8
    rem = n_nodes - rpt * NS
    mesh = plsc.VectorSubcoreMesh(core_axis_name="c", subcore_axis_name="s")
    cp = pltpu.CompilerParams()
    if "needs_layout_passes" in pltpu.CompilerParams.__dataclass_fields__:
        cp = dataclasses.replace(cp, needs_layout_passes=False)

    @functools.partial(
        pl.kernel,
        out_type=jax.ShapeDtypeStruct((NC, n_nodes, d), jnp.float32),
        mesh=mesh,
        compiler_params=cp,
        scratch_types=[
            pltpu.VMEM((CH,), jnp.int32),
            pltpu.VMEM((CH,), jnp.int32),
            pltpu.VMEM((CH,), jnp.float32),
            pltpu.VMEM((CH, d), jnp.float32),
            pltpu.VMEM_SHARED((n_nodes, d), jnp.float32),
        ],
    )
    def spmm(emb, srcs, dsts, ws, zeros, out, src_v, dst_v, w_v, rows_v, acc):
        c = lax.axis_index("c")
        s = lax.axis_index("s")
        zb = s * rpt
        pltpu.sync_copy(zeros.at[pl.ds(zb, rpt)], acc.at[pl.ds(zb, rpt)])
        if rem:
            @pl.when(s == NS - 1)
            def _():
                pltpu.sync_copy(zeros.at[pl.ds(NS * rpt, rem)],
                                acc.at[pl.ds(NS * rpt, rem)])
        plsc.subcore_barrier()

        wid = s * NC + c
        ebase = wid * (t_chunks * CH)

        @pl.loop(0, t_chunks)
        def _chunk(t):
            base = ebase + t * CH
            pltpu.sync_copy(srcs.at[pl.ds(base, CH)], src_v)
            pltpu.sync_copy(dsts.at[pl.ds(base, CH)], dst_v)
            pltpu.sync_copy(ws.at[pl.ds(base, CH)], w_v)
            pltpu.sync_copy(emb.at[src_v], rows_v)

            @pl.loop(0, CH)
            def _edge(e):
                idx16 = jnp.full((L,), 0, jnp.int32) + e
                w16 = plsc.load_gather(w_v, [idx16])
                for k in range(d // L):
                    sl = pl.ds(k * L, L)
                    rows_v[e, sl] = rows_v[e, sl] * w16

            pltpu.sync_copy(rows_v, acc.at[dst_v], add=True)

        plsc.subcore_barrier()
        pltpu.sync_copy(acc.at[pl.ds(zb, rpt)], out.at[c, pl.ds(zb, rpt)])
        if rem:
            @pl.when(s == NS - 1)
            def _():
                pltpu.sync_copy(acc.at[pl.ds(NS * rpt, rem)],
                                out.at[c, pl.ds(NS * rpt, rem)])

    return spmm


def kernel(user_fea, item_fea, global_embedding_u, global_embedding_i, W, b,
           edge_weight, edge_index):
    nu, d = user_fea.shape
    ni = item_fea.shape[0]
    n = nu + ni
    e = edge_weight.shape[0]

    ego = _embed_call(user_fea, item_fea, global_embedding_u,
                      global_embedding_i, W, b)

    per = NC * NS * CH
    t_chunks = -(-e // per)
    pad = t_chunks * per - e
    dst = edge_index[0]
    src = edge_index[1]
    w = edge_weight
    if pad:
        fill = (jnp.arange(pad, dtype=jnp.int32) % n).astype(jnp.int32)
        src = jnp.concatenate([src, fill])
        dst = jnp.concatenate([dst, fill])
        w = jnp.concatenate([w, jnp.zeros((pad,), jnp.float32)])
    zeros = jnp.zeros((n, d), jnp.float32)

    spmm = _make_spmm(n, d, t_chunks)
    all_emb = ego
    ui = ego
    for _ in range(N_LAYERS):
        parts = spmm(all_emb, src, dst, w, zeros)
        all_emb, ui = _cosine_call(parts, ego, ui)
    return ui[:nu], ui[nu:]


# speedup vs baseline: 8.8504x; 2.2856x over previous
import dataclasses
import functools

import jax
import jax.numpy as jnp
from jax import lax
from jax.experimental import pallas as pl
from jax.experimental.pallas import tpu as pltpu
from jax.experimental.pallas import tpu_sc as plsc

NC = 2
NS = 16
L = 16
CH = 64
N_LAYERS = 4


def _embed_call(user_fea, item_fea, gu, gi, W, b):
    nu, d = user_fea.shape
    ni = item_fea.shape[0]

    def body(uf, itf, gu_r, gi_r, w_r, b_r, out):
        gus = jnp.sum(gu_r[...], axis=0)
        gis = jnp.sum(gi_r[...], axis=0)
        out[:nu] = uf[...] + gus[None, :]
        acts = jnp.dot(itf[...], w_r[...],
                       preferred_element_type=jnp.float32,
                       precision=lax.Precision.HIGHEST)
        out[nu:] = jnp.tanh(acts + b_r[...][None, :]) + gis[None, :]

    return pl.pallas_call(
        body,
        out_shape=jax.ShapeDtypeStruct((nu + ni, d), jnp.float32),
    )(user_fea, item_fea, gu, gi, W, b)


def _cosine_call(parts, ego, ui):
    n, d = ego.shape

    def body(p, e_r, u_r, new_r, uo_r):
        x = p[0] + p[1]
        e = e_r[...]
        num = jnp.sum(x * e, axis=-1, keepdims=True)
        nx = jnp.sqrt(jnp.sum(x * x, axis=-1, keepdims=True))
        ne = jnp.sqrt(jnp.sum(e * e, axis=-1, keepdims=True))
        wgt = num / jnp.maximum(nx * ne, 1e-8)
        xw = x * wgt
        new_r[...] = xw
        uo_r[...] = u_r[...] + xw

    return pl.pallas_call(
        body,
        out_shape=(jax.ShapeDtypeStruct((n, d), jnp.float32),
                   jax.ShapeDtypeStruct((n, d), jnp.float32)),
    )(parts, ego, ui)


NBUF = 4


@functools.lru_cache(maxsize=None)
def _make_spmm(n_nodes, d, t_chunks):
    assert t_chunks % NBUF == 0 and t_chunks >= 2 * NBUF
    rpt = (n_nodes // (NS * 8)) * 8
    rem = n_nodes - rpt * NS
    mesh = plsc.VectorSubcoreMesh(core_axis_name="c", subcore_axis_name="s")
    cp = pltpu.CompilerParams()
    if "needs_layout_passes" in pltpu.CompilerParams.__dataclass_fields__:
        cp = dataclasses.replace(cp, needs_layout_passes=False)

    scratch = []
    for _ in range(NBUF):
        scratch += [pltpu.VMEM((CH,), jnp.int32),
                    pltpu.VMEM((CH,), jnp.int32),
                    pltpu.VMEM((CH,), jnp.float32),
                    pltpu.VMEM((CH, d), jnp.float32)]
    scratch += [pltpu.SemaphoreType.DMA] * (3 * NBUF)
    scratch += [pltpu.VMEM_SHARED((n_nodes, d), jnp.float32)]

    @functools.partial(
        pl.kernel,
        out_type=jax.ShapeDtypeStruct((NC, n_nodes, d), jnp.float32),
        mesh=mesh,
        compiler_params=cp,
        scratch_types=scratch,
    )
    def spmm(emb, srcs, dsts, ws, zeros, out, *sc):
        bufs = [sc[4 * i:4 * i + 4] for i in range(NBUF)]
        i_sem = sc[4 * NBUF:4 * NBUF + NBUF]
        g_sem = sc[5 * NBUF:5 * NBUF + NBUF]
        s_sem = sc[6 * NBUF:6 * NBUF + NBUF]
        acc = sc[7 * NBUF]

        c = lax.axis_index("c")
        s = lax.axis_index("s")
        zb = s * rpt
        pltpu.sync_copy(zeros.at[pl.ds(zb, rpt)], acc.at[pl.ds(zb, rpt)])
        if rem:
            @pl.when(s == NS - 1)
            def _():
                pltpu.sync_copy(zeros.at[pl.ds(NS * rpt, rem)],
                                acc.at[pl.ds(NS * rpt, rem)])
        plsc.subcore_barrier()

        wid = s * NC + c
        ebase = wid * (t_chunks * CH)

        def issue_idx(t, b):
            base = ebase + t * CH
            sv, dv, wv, _ = bufs[b]
            pltpu.async_copy(srcs.at[pl.ds(base, CH)], sv, i_sem[b])
            pltpu.async_copy(dsts.at[pl.ds(base, CH)], dv, i_sem[b])
            pltpu.async_copy(ws.at[pl.ds(base, CH)], wv, i_sem[b])

        def wait_idx(b):
            sv, dv, wv, _ = bufs[b]
            pltpu.make_async_copy(srcs.at[pl.ds(0, CH)], sv, i_sem[b]).wait()
            pltpu.make_async_copy(dsts.at[pl.ds(0, CH)], dv, i_sem[b]).wait()
            pltpu.make_async_copy(ws.at[pl.ds(0, CH)], wv, i_sem[b]).wait()

        def issue_gather(b):
            sv, _, _, rv = bufs[b]
            pltpu.async_copy(emb.at[sv], rv, g_sem[b])

        def wait_gather(b):
            sv, _, _, rv = bufs[b]
            pltpu.make_async_copy(emb.at[sv], rv, g_sem[b]).wait()

        def issue_scatter(b):
            _, dv, _, rv = bufs[b]
            pltpu.async_copy(rv, acc.at[dv], s_sem[b], add=True)

        def wait_scatter(b):
            _, dv, _, rv = bufs[b]
            pltpu.make_async_copy(rv, acc.at[dv], s_sem[b]).wait()

        def multiply(b):
            _, _, wv, rv = bufs[b]

            @pl.loop(0, CH, step=4)
            def _edge(e0):
                for u in range(4):
                    e = e0 + u
                    w16 = plsc.load_gather(wv, [jnp.full((L,), 0, jnp.int32) + e])
                    for k in range(d // L):
                        sl = pl.ds(k * L, L)
                        rv[e, sl] = rv[e, sl] * w16

        issue_idx(0, 0)
        issue_idx(1, 1)
        wait_idx(0)
        issue_gather(0)

        @pl.loop(0, t_chunks, step=NBUF)
        def _outer(t0):
            for b in range(NBUF):
                t = t0 + b
                b2 = (b + 2) % NBUF
                b1 = (b + 1) % NBUF

                @pl.when(t + 2 < t_chunks)
                def _():
                    @pl.when(t >= 2)
                    def _():
                        wait_scatter(b2)
                    issue_idx(t + 2, b2)

                @pl.when(t + 1 < t_chunks)
                def _():
                    wait_idx(b1)
                    issue_gather(b1)

                wait_gather(b)
                multiply(b)
                issue_scatter(b)

        for b in range(NBUF):
            wait_scatter(b)

        plsc.subcore_barrier()
        pltpu.sync_copy(acc.at[pl.ds(zb, rpt)], out.at[c, pl.ds(zb, rpt)])
        if rem:
            @pl.when(s == NS - 1)
            def _():
                pltpu.sync_copy(acc.at[pl.ds(NS * rpt, rem)],
                                out.at[c, pl.ds(NS * rpt, rem)])

    return spmm


def kernel(user_fea, item_fea, global_embedding_u, global_embedding_i, W, b,
           edge_weight, edge_index):
    nu, d = user_fea.shape
    ni = item_fea.shape[0]
    n = nu + ni
    e = edge_weight.shape[0]

    ego = _embed_call(user_fea, item_fea, global_embedding_u,
                      global_embedding_i, W, b)

    per = NC * NS * CH
    t_chunks = -(-e // per)
    t_chunks += (-t_chunks) % NBUF
    t_chunks = max(t_chunks, 2 * NBUF)
    pad = t_chunks * per - e
    dst = edge_index[0]
    src = edge_index[1]
    w = edge_weight
    if pad:
        fill = (jnp.arange(pad, dtype=jnp.int32) % n).astype(jnp.int32)
        src = jnp.concatenate([src, fill])
        dst = jnp.concatenate([dst, fill])
        w = jnp.concatenate([w, jnp.zeros((pad,), jnp.float32)])
    zeros = jnp.zeros((n, d), jnp.float32)

    spmm = _make_spmm(n, d, t_chunks)
    all_emb = ego
    ui = ego
    for _ in range(N_LAYERS):
        parts = spmm(all_emb, src, dst, w, zeros)
        all_emb, ui = _cosine_call(parts, ego, ui)
    return ui[:nu], ui[nu:]


# speedup vs baseline: 9.8536x; 1.1133x over previous
import dataclasses
import functools

import jax
import jax.numpy as jnp
from jax import lax
from jax.experimental import pallas as pl
from jax.experimental.pallas import tpu as pltpu
from jax.experimental.pallas import tpu_sc as plsc

NC = 2
NS = 16
L = 16
CH = 64
N_LAYERS = 4


def _embed_call(user_fea, item_fea, gu, gi, W, b):
    nu, d = user_fea.shape
    ni = item_fea.shape[0]

    def body(uf, itf, gu_r, gi_r, w_r, b_r, out):
        gus = jnp.sum(gu_r[...], axis=0)
        gis = jnp.sum(gi_r[...], axis=0)
        out[:nu] = uf[...] + gus[None, :]
        acts = jnp.dot(itf[...], w_r[...],
                       preferred_element_type=jnp.float32,
                       precision=lax.Precision.HIGHEST)
        out[nu:] = jnp.tanh(acts + b_r[...][None, :]) + gis[None, :]

    return pl.pallas_call(
        body,
        out_shape=jax.ShapeDtypeStruct((nu + ni, d), jnp.float32),
    )(user_fea, item_fea, gu, gi, W, b)


def _cosine_call(parts, ego, ui):
    n, d = ego.shape

    def body(p, e_r, u_r, new_r, uo_r):
        x = p[0] + p[1]
        e = e_r[...]
        num = jnp.sum(x * e, axis=-1, keepdims=True)
        nx = jnp.sqrt(jnp.sum(x * x, axis=-1, keepdims=True))
        ne = jnp.sqrt(jnp.sum(e * e, axis=-1, keepdims=True))
        wgt = num / jnp.maximum(nx * ne, 1e-8)
        xw = x * wgt
        new_r[...] = xw
        uo_r[...] = u_r[...] + xw

    return pl.pallas_call(
        body,
        out_shape=(jax.ShapeDtypeStruct((n, d), jnp.float32),
                   jax.ShapeDtypeStruct((n, d), jnp.float32)),
    )(parts, ego, ui)


NBUF = 4


@functools.lru_cache(maxsize=None)
def _make_spmm(n_nodes, d, t_chunks):
    assert t_chunks % NBUF == 0 and t_chunks >= 2 * NBUF
    rpt = (n_nodes // (NS * 8)) * 8
    rem = n_nodes - rpt * NS
    mesh = plsc.VectorSubcoreMesh(core_axis_name="c", subcore_axis_name="s")
    cp = pltpu.CompilerParams()
    if "needs_layout_passes" in pltpu.CompilerParams.__dataclass_fields__:
        cp = dataclasses.replace(cp, needs_layout_passes=False)

    scratch = []
    for _ in range(NBUF):
        scratch += [pltpu.VMEM((CH,), jnp.int32),
                    pltpu.VMEM((CH,), jnp.int32),
                    pltpu.VMEM((CH,), jnp.float32),
                    pltpu.VMEM((CH, d), jnp.float32)]
    scratch += [pltpu.SemaphoreType.DMA] * (3 * NBUF)
    scratch += [pltpu.VMEM_SHARED((n_nodes, d), jnp.float32)]

    @functools.partial(
        pl.kernel,
        out_type=jax.ShapeDtypeStruct((NC, n_nodes, d), jnp.float32),
        mesh=mesh,
        compiler_params=cp,
        scratch_types=scratch,
    )
    def spmm(emb, srcs, dsts, ws, zeros, out, *sc):
        bufs = [sc[4 * i:4 * i + 4] for i in range(NBUF)]
        i_sem = sc[4 * NBUF:4 * NBUF + NBUF]
        g_sem = sc[5 * NBUF:5 * NBUF + NBUF]
        s_sem = sc[6 * NBUF:6 * NBUF + NBUF]
        acc = sc[7 * NBUF]

        c = lax.axis_index("c")
        s = lax.axis_index("s")
        zb = s * rpt
        pltpu.sync_copy(zeros.at[pl.ds(zb, rpt)], acc.at[pl.ds(zb, rpt)])
        if rem:
            @pl.when(s == NS - 1)
            def _():
                pltpu.sync_copy(zeros.at[pl.ds(NS * rpt, rem)],
                                acc.at[pl.ds(NS * rpt, rem)])
        plsc.subcore_barrier()

        wid = s * NC + c
        ebase = wid * (t_chunks * CH)

        def issue_idx(t, b):
            base = ebase + t * CH
            sv, dv, wv, _ = bufs[b]
            pltpu.async_copy(srcs.at[pl.ds(base, CH)], sv, i_sem[b])
            pltpu.async_copy(dsts.at[pl.ds(base, CH)], dv, i_sem[b])
            pltpu.async_copy(ws.at[pl.ds(base, CH)], wv, i_sem[b])

        def wait_idx(b):
            sv, dv, wv, _ = bufs[b]
            pltpu.make_async_copy(srcs.at[pl.ds(0, CH)], sv, i_sem[b]).wait()
            pltpu.make_async_copy(dsts.at[pl.ds(0, CH)], dv, i_sem[b]).wait()
            pltpu.make_async_copy(ws.at[pl.ds(0, CH)], wv, i_sem[b]).wait()

        def issue_gather(b):
            sv, _, _, rv = bufs[b]
            pltpu.async_copy(emb.at[sv], rv, g_sem[b])

        def wait_gather(b):
            sv, _, _, rv = bufs[b]
            pltpu.make_async_copy(emb.at[sv], rv, g_sem[b]).wait()

        def issue_scatter(b):
            _, dv, _, rv = bufs[b]
            pltpu.async_copy(rv, acc.at[dv], s_sem[b], add=True)

        def wait_scatter(b):
            _, dv, _, rv = bufs[b]
            pltpu.make_async_copy(rv, acc.at[dv], s_sem[b]).wait()

        lane_consts = [jnp.full((L,), u, jnp.int32) for u in range(L)]

        def multiply(b):
            _, _, wv, rv = bufs[b]

            @pl.loop(0, CH, step=L)
            def _edge(e0):
                wvec = wv[pl.ds(e0, L)]
                for u in range(L):
                    w16 = wvec.at[lane_consts[u]].get(mode="promise_in_bounds")
                    e = e0 + u
                    for k in range(d // L):
                        sl = pl.ds(k * L, L)
                        rv[e, sl] = rv[e, sl] * w16

        issue_idx(0, 0)
        issue_idx(1, 1)
        wait_idx(0)
        issue_gather(0)

        @pl.loop(0, t_chunks, step=NBUF)
        def _outer(t0):
            for b in range(NBUF):
                t = t0 + b
                b2 = (b + 2) % NBUF
                b1 = (b + 1) % NBUF

                @pl.when(t + 2 < t_chunks)
                def _():
                    @pl.when(t >= 2)
                    def _():
                        wait_scatter(b2)
                    issue_idx(t + 2, b2)

                @pl.when(t + 1 < t_chunks)
                def _():
                    wait_idx(b1)
                    issue_gather(b1)

                wait_gather(b)
                multiply(b)
                issue_scatter(b)

        for b in range(NBUF):
            wait_scatter(b)

        plsc.subcore_barrier()
        pltpu.sync_copy(acc.at[pl.ds(zb, rpt)], out.at[c, pl.ds(zb, rpt)])
        if rem:
            @pl.when(s == NS - 1)
            def _():
                pltpu.sync_copy(acc.at[pl.ds(NS * rpt, rem)],
                                out.at[c, pl.ds(NS * rpt, rem)])

    return spmm


def kernel(user_fea, item_fea, global_embedding_u, global_embedding_i, W, b,
           edge_weight, edge_index):
    nu, d = user_fea.shape
    ni = item_fea.shape[0]
    n = nu + ni
    e = edge_weight.shape[0]

    ego = _embed_call(user_fea, item_fea, global_embedding_u,
                      global_embedding_i, W, b)

    per = NC * NS * CH
    t_chunks = -(-e // per)
    t_chunks += (-t_chunks) % NBUF
    t_chunks = max(t_chunks, 2 * NBUF)
    pad = t_chunks * per - e
    dst = edge_index[0]
    src = edge_index[1]
    w = edge_weight
    if pad:
        fill = (jnp.arange(pad, dtype=jnp.int32) % n).astype(jnp.int32)
        src = jnp.concatenate([src, fill])
        dst = jnp.concatenate([dst, fill])
        w = jnp.concatenate([w, jnp.zeros((pad,), jnp.float32)])
    zeros = jnp.zeros((n, d), jnp.float32)

    spmm = _make_spmm(n, d, t_chunks)
    all_emb = ego
    ui = ego
    for _ in range(N_LAYERS):
        parts = spmm(all_emb, src, dst, w, zeros)
        all_emb, ui = _cosine_call(parts, ego, ui)
    return ui[:nu], ui[nu:]


# speedup vs baseline: 10.5002x; 1.0656x over previous
import dataclasses
import functools

import jax
import jax.numpy as jnp
from jax import lax
from jax.experimental import pallas as pl
from jax.experimental.pallas import tpu as pltpu
from jax.experimental.pallas import tpu_sc as plsc

NC = 2
NS = 16
L = 16
CH = 96
N_LAYERS = 4


def _embed_call(user_fea, item_fea, gu, gi, W, b):
    nu, d = user_fea.shape
    ni = item_fea.shape[0]

    def body(uf, itf, gu_r, gi_r, w_r, b_r, out):
        gus = jnp.sum(gu_r[...], axis=0)
        gis = jnp.sum(gi_r[...], axis=0)
        out[:nu] = uf[...] + gus[None, :]
        acts = jnp.dot(itf[...], w_r[...],
                       preferred_element_type=jnp.float32,
                       precision=lax.Precision.HIGHEST)
        out[nu:] = jnp.tanh(acts + b_r[...][None, :]) + gis[None, :]

    return pl.pallas_call(
        body,
        out_shape=jax.ShapeDtypeStruct((nu + ni, d), jnp.float32),
    )(user_fea, item_fea, gu, gi, W, b)


def _cosine_call(parts, ego, ui):
    n, d = ego.shape

    def body(p, e_r, u_r, new_r, uo_r):
        x = p[0] + p[1]
        e = e_r[...]
        num = jnp.sum(x * e, axis=-1, keepdims=True)
        nx = jnp.sqrt(jnp.sum(x * x, axis=-1, keepdims=True))
        ne = jnp.sqrt(jnp.sum(e * e, axis=-1, keepdims=True))
        wgt = num / jnp.maximum(nx * ne, 1e-8)
        xw = x * wgt
        new_r[...] = xw
        uo_r[...] = u_r[...] + xw

    return pl.pallas_call(
        body,
        out_shape=(jax.ShapeDtypeStruct((n, d), jnp.float32),
                   jax.ShapeDtypeStruct((n, d), jnp.float32)),
    )(parts, ego, ui)


NBUF = 4


@functools.lru_cache(maxsize=None)
def _make_spmm(n_nodes, d, t_chunks):
    assert t_chunks % NBUF == 0 and t_chunks >= 2 * NBUF
    rpt = (n_nodes // (NS * 8)) * 8
    rem = n_nodes - rpt * NS
    mesh = plsc.VectorSubcoreMesh(core_axis_name="c", subcore_axis_name="s")
    cp = pltpu.CompilerParams()
    if "needs_layout_passes" in pltpu.CompilerParams.__dataclass_fields__:
        cp = dataclasses.replace(cp, needs_layout_passes=False)

    scratch = []
    for _ in range(NBUF):
        scratch += [pltpu.VMEM((CH,), jnp.int32),
                    pltpu.VMEM((CH,), jnp.int32),
                    pltpu.VMEM((CH,), jnp.float32),
                    pltpu.VMEM((CH, d), jnp.float32)]
    scratch += [pltpu.SemaphoreType.DMA] * (3 * NBUF)
    scratch += [pltpu.VMEM_SHARED((n_nodes, d), jnp.float32)]

    @functools.partial(
        pl.kernel,
        out_type=jax.ShapeDtypeStruct((NC, n_nodes, d), jnp.float32),
        mesh=mesh,
        compiler_params=cp,
        scratch_types=scratch,
    )
    def spmm(emb, srcs, dsts, ws, zeros, out, *sc):
        bufs = [sc[4 * i:4 * i + 4] for i in range(NBUF)]
        i_sem = sc[4 * NBUF:4 * NBUF + NBUF]
        g_sem = sc[5 * NBUF:5 * NBUF + NBUF]
        s_sem = sc[6 * NBUF:6 * NBUF + NBUF]
        acc = sc[7 * NBUF]

        c = lax.axis_index("c")
        s = lax.axis_index("s")
        zb = s * rpt
        wid = s * NC + c
        ebase = wid * (t_chunks * CH)

        def issue_idx(t, b):
            base = ebase + t * CH
            sv, dv, wv, _ = bufs[b]
            pltpu.async_copy(srcs.at[pl.ds(base, CH)], sv, i_sem[b])
            pltpu.async_copy(dsts.at[pl.ds(base, CH)], dv, i_sem[b])
            pltpu.async_copy(ws.at[pl.ds(base, CH)], wv, i_sem[b])

        def wait_idx(b):
            sv, dv, wv, _ = bufs[b]
            pltpu.make_async_copy(srcs.at[pl.ds(0, CH)], sv, i_sem[b]).wait()
            pltpu.make_async_copy(dsts.at[pl.ds(0, CH)], dv, i_sem[b]).wait()
            pltpu.make_async_copy(ws.at[pl.ds(0, CH)], wv, i_sem[b]).wait()

        def issue_gather(b):
            sv, _, _, rv = bufs[b]
            pltpu.async_copy(emb.at[sv], rv, g_sem[b])

        def wait_gather(b):
            sv, _, _, rv = bufs[b]
            pltpu.make_async_copy(emb.at[sv], rv, g_sem[b]).wait()

        def issue_scatter(b):
            _, dv, _, rv = bufs[b]
            pltpu.async_copy(rv, acc.at[dv], s_sem[b], add=True)

        def wait_scatter(b):
            _, dv, _, rv = bufs[b]
            pltpu.make_async_copy(rv, acc.at[dv], s_sem[b]).wait()

        lane_consts = [jnp.full((L,), u, jnp.int32) for u in range(L)]

        def multiply(b):
            _, _, wv, rv = bufs[b]

            @pl.loop(0, CH, step=L)
            def _edge(e0):
                wvec = wv[pl.ds(e0, L)]
                for u in range(L):
                    w16 = wvec.at[lane_consts[u]].get(mode="promise_in_bounds")
                    e = e0 + u
                    for k in range(d // L):
                        sl = pl.ds(k * L, L)
                        rv[e, sl] = rv[e, sl] * w16

        issue_idx(0, 0)
        issue_idx(1, 1)
        pltpu.sync_copy(zeros.at[pl.ds(zb, rpt)], acc.at[pl.ds(zb, rpt)])
        if rem:
            @pl.when(s == NS - 1)
            def _():
                pltpu.sync_copy(zeros.at[pl.ds(NS * rpt, rem)],
                                acc.at[pl.ds(NS * rpt, rem)])
        plsc.subcore_barrier()
        wait_idx(0)
        issue_gather(0)

        @pl.loop(0, t_chunks, step=NBUF)
        def _outer(t0):
            for b in range(NBUF):
                t = t0 + b
                b2 = (b + 2) % NBUF
                b1 = (b + 1) % NBUF

                @pl.when(t + 2 < t_chunks)
                def _():
                    @pl.when(t >= 2)
                    def _():
                        wait_scatter(b2)
                    issue_idx(t + 2, b2)

                @pl.when(t + 1 < t_chunks)
                def _():
                    wait_idx(b1)
                    issue_gather(b1)

                wait_gather(b)
                multiply(b)
                issue_scatter(b)

        for b in range(NBUF):
            wait_scatter(b)

        plsc.subcore_barrier()
        pltpu.sync_copy(acc.at[pl.ds(zb, rpt)], out.at[c, pl.ds(zb, rpt)])
        if rem:
            @pl.when(s == NS - 1)
            def _():
                pltpu.sync_copy(acc.at[pl.ds(NS * rpt, rem)],
                                out.at[c, pl.ds(NS * rpt, rem)])

    return spmm


def kernel(user_fea, item_fea, global_embedding_u, global_embedding_i, W, b,
           edge_weight, edge_index):
    nu, d = user_fea.shape
    ni = item_fea.shape[0]
    n = nu + ni
    e = edge_weight.shape[0]

    ego = _embed_call(user_fea, item_fea, global_embedding_u,
                      global_embedding_i, W, b)

    per = NC * NS * CH
    t_chunks = -(-e // per)
    t_chunks += (-t_chunks) % NBUF
    t_chunks = max(t_chunks, 2 * NBUF)
    pad = t_chunks * per - e
    dst = edge_index[0]
    src = edge_index[1]
    w = edge_weight
    if pad:
        fill = (jnp.arange(pad, dtype=jnp.int32) % n).astype(jnp.int32)
        src = jnp.concatenate([src, fill])
        dst = jnp.concatenate([dst, fill])
        w = jnp.concatenate([w, jnp.zeros((pad,), jnp.float32)])
    zeros = jnp.zeros((n, d), jnp.float32)

    spmm = _make_spmm(n, d, t_chunks)
    all_emb = ego
    ui = ego
    for _ in range(N_LAYERS):
        parts = spmm(all_emb, src, dst, w, zeros)
        all_emb, ui = _cosine_call(parts, ego, ui)
    return ui[:nu], ui[nu:]
